# baseline (speedup 1.0000x reference)
import jax
import jax.numpy as jnp
from jax.experimental import pallas as pl
from jax.experimental.pallas import tpu as pltpu


def _body(sidx_ref, ts_ref, ss_ref, x_ref, temb_ref, semb_ref, o_ref):
    t = pl.program_id(1)
    idx = sidx_ref[0] + t
    trow = temb_ref[pl.ds(idx, 1), :]
    pos = ts_ref[0] * trow + ss_ref[0] * semb_ref[...]
    o_ref[0, 0] = x_ref[0, 0] + pos


def kernel(x, temporal_embed, spatial_embed, temporal_scale, spatial_scale, start_idx):
    B, T, S, D = x.shape
    sidx = jnp.asarray(start_idx, jnp.int32).reshape(1)
    smem = pl.BlockSpec(memory_space=pltpu.SMEM)
    grid = (B, T)
    return pl.pallas_call(
        _body,
        grid=grid,
        in_specs=[
            smem,
            smem,
            smem,
            pl.BlockSpec((1, 1, S, D), lambda b, t: (b, t, 0, 0)),
            pl.BlockSpec((temporal_embed.shape[0], D), lambda b, t: (0, 0)),
            pl.BlockSpec((S, D), lambda b, t: (0, 0)),
        ],
        out_specs=pl.BlockSpec((1, 1, S, D), lambda b, t: (b, t, 0, 0)),
        out_shape=jax.ShapeDtypeStruct((B, T, S, D), x.dtype),
    )(sidx, temporal_scale, spatial_scale, x, temporal_embed, spatial_embed)
